# baseline (speedup 1.0000x reference)
import functools

import jax
import jax.numpy as jnp
from jax import lax
from jax.experimental import pallas as pl
from jax.experimental.pallas import tpu as pltpu
from jax.experimental.pallas import tpu_sc as plsc

_N = 10000
_E = 320000
_L = 100000
_D = 128
_H = 128
_C = 64

_NC = 2
_NS = 16
_NW = _NC * _NS

_NP = 10240
_RPT = _NP // _NS

_EPW = _E // _NW
_ECB = 96
_EFULL = 104
_ETAIL = _EFULL * _ECB

_CNTW = 16

_LP = 100352
_LPW = _LP // _NW
_LC = 64
_LNC = _LPW // _LC

_ROWB = 25
_RB = _N // _ROWB


def _dotT(x, w):
    return lax.dot_general(x, w, (((1,), (1,)), ((), ())),
                           preferred_element_type=jnp.float32)


def _tc_pre(x, w1l, w1r, b1):
    def body(x_ref, wl_ref, wr_ref, b_ref, y_ref, h_ref):
        xb = x_ref[...]
        y_ref[...] = _dotT(xb, wl_ref[...])
        h_ref[...] = _dotT(xb, wr_ref[...]) + b_ref[...]
    return pl.pallas_call(
        body,
        grid=(_ROWB,),
        in_specs=[
            pl.BlockSpec((_RB, _D), lambda i: (i, 0)),
            pl.BlockSpec((_H, _D), lambda i: (0, 0)),
            pl.BlockSpec((_H, _D), lambda i: (0, 0)),
            pl.BlockSpec((1, _H), lambda i: (0, 0)),
        ],
        out_specs=[pl.BlockSpec((_RB, _H), lambda i: (i, 0))] * 2,
        out_shape=[jax.ShapeDtypeStruct((_N, _H), jnp.float32)] * 2,
    )(x, w1l, w1r, b1.reshape(1, _H))


def _tc_mid(s1, cnt, h1, w2lp, w2r, b2):
    def body(s_ref, c_ref, h_ref, wl_ref, wr_ref, b_ref, y_ref, h2_ref):
        s = s_ref[0] + s_ref[1]
        c = c_ref[:, 0:1] + c_ref[:, 1:2]
        x1 = jnp.maximum(s / jnp.maximum(c, 1.0) + h_ref[...], 0.0)
        y_ref[...] = _dotT(x1, wl_ref[...])
        h2_ref[...] = _dotT(x1, wr_ref[...]) + b_ref[...]
    return pl.pallas_call(
        body,
        grid=(_ROWB,),
        in_specs=[
            pl.BlockSpec((_NC, _RB, _H), lambda i: (0, i, 0)),
            pl.BlockSpec((_RB, _NC), lambda i: (i, 0)),
            pl.BlockSpec((_RB, _H), lambda i: (i, 0)),
            pl.BlockSpec((_H, _H), lambda i: (0, 0)),
            pl.BlockSpec((_C, _H), lambda i: (0, 0)),
            pl.BlockSpec((1, _C), lambda i: (0, 0)),
        ],
        out_specs=[pl.BlockSpec((_RB, _H), lambda i: (i, 0)),
                   pl.BlockSpec((_RB, _C), lambda i: (i, 0))],
        out_shape=[jax.ShapeDtypeStruct((_N, _H), jnp.float32),
                   jax.ShapeDtypeStruct((_N, _C), jnp.float32)],
    )(s1, cnt, h1, w2lp, w2r, b2.reshape(1, _C))


def _tc_post(s2, cnt, h2):
    def body(s_ref, c_ref, h_ref, x_ref):
        s = s_ref[0][:, 0:_C] + s_ref[1][:, 0:_C]
        c = c_ref[:, 0:1] + c_ref[:, 1:2]
        x2 = s / jnp.maximum(c, 1.0) + h_ref[...]
        x_ref[...] = jnp.concatenate(
            [x2, jnp.zeros((_RB, _H - _C), jnp.float32)], axis=1)
    return pl.pallas_call(
        body,
        grid=(_ROWB,),
        in_specs=[
            pl.BlockSpec((_NC, _RB, _H), lambda i: (0, i, 0)),
            pl.BlockSpec((_RB, _NC), lambda i: (i, 0)),
            pl.BlockSpec((_RB, _C), lambda i: (i, 0)),
        ],
        out_specs=pl.BlockSpec((_RB, _H), lambda i: (i, 0)),
        out_shape=jax.ShapeDtypeStruct((_N, _H), jnp.float32),
    )(s2, cnt, h2)


def _zero_vmem(ref, nrow, width):
    def zrow(i, _):
        for q in range(width // 16):
            ref[i, pl.ds(q * 16, 16)] = jnp.zeros((16,), jnp.float32)
        return 0
    lax.fori_loop(0, nrow, zrow, 0)


def _make_agg(F, with_cnt):
    mesh = plsc.VectorSubcoreMesh(core_axis_name="c", subcore_axis_name="s")
    out_type = [jax.ShapeDtypeStruct((_NC * _NP, F), jnp.float32)]
    scratch = [
        pltpu.VMEM((_EPW // 2,), jnp.int32),
        pltpu.VMEM((_EPW // 2,), jnp.int32),
        pltpu.VMEM((_ECB,), jnp.int32),
        pltpu.VMEM((_ECB,), jnp.int32),
        pltpu.VMEM((_ECB,), jnp.int32),
        pltpu.VMEM((_ECB,), jnp.int32),
        pltpu.VMEM((16,), jnp.int32),
        pltpu.VMEM((16,), jnp.int32),
        pltpu.VMEM((_ECB, F), jnp.float32),
        pltpu.VMEM((_ECB, F), jnp.float32),
        pltpu.VMEM_SHARED((_NP, F), jnp.float32),
        pltpu.SemaphoreType.DMA,
        pltpu.SemaphoreType.DMA,
    ]
    if with_cnt:
        out_type.append(jax.ShapeDtypeStruct((_NC * _NP,), jnp.float32))
        scratch += [
            pltpu.VMEM((_ECB,), jnp.float32),
            pltpu.VMEM_SHARED((_NP,), jnp.float32),
        ]

    def body(y_hbm, src_hbm, dst_hbm, src16_hbm, dst16_hbm, *rest):
        if with_cnt:
            (out_hbm, cnt_hbm, srcall_v, dstall_v, src0_v, src1_v,
             dst0_v, dst1_v, srct_v, dstt_v,
             rows0, rows1, agg_sh, semg0, semg1, ones_v, cnt_sh) = rest
        else:
            (out_hbm, srcall_v, dstall_v, src0_v, src1_v,
             dst0_v, dst1_v, srct_v, dstt_v,
             rows0, rows1, agg_sh, semg0, semg1) = rest
        cid = lax.axis_index("c")
        sid = lax.axis_index("s")
        wid = cid * _NS + sid
        ebase = wid * _EPW

        ebase2 = pl.multiple_of(wid * (_EPW // 2), 8)
        cpa = pltpu.async_copy(
            src16_hbm.at[pl.ds(ebase2, _EPW // 2)], srcall_v, semg0)
        cpb = pltpu.async_copy(
            dst16_hbm.at[pl.ds(ebase2, _EPW // 2)], dstall_v, semg0)

        _zero_vmem(rows0, 80, F)
        if with_cnt:
            for q in range(80 // 16):
                ones_v[pl.ds(q * 16, 16)] = jnp.zeros((16,), jnp.float32)

        def zslab(k, _):
            r0 = sid * _RPT + k * 80
            pltpu.sync_copy(rows0.at[pl.ds(0, 80)], agg_sh.at[pl.ds(r0, 80)])
            if with_cnt:
                pltpu.sync_copy(ones_v.at[pl.ds(0, 80)], cnt_sh.at[pl.ds(r0, 80)])
            return 0
        lax.fori_loop(0, _RPT // 80, zslab, 0)

        if with_cnt:
            for q in range(_ECB // 16):
                ones_v[pl.ds(q * 16, 16)] = jnp.ones((16,), jnp.float32)

        cpa.wait()
        cpb.wait()
        plsc.subcore_barrier()

        def prep_idx(j, sbuf, dbuf):
            for q in range(_ECB // 32):
                off = j * (_ECB // 2) + q * 16
                sv = plsc.bitcast(srcall_v[pl.ds(off, 16)], jnp.int16)
                s0, s1 = plsc.unpack(sv, format=plsc.PackFormat.INTERLEAVED)
                sbuf[pl.ds(q * 32, 16)] = s0
                sbuf[pl.ds(q * 32 + 16, 16)] = s1
                dv = plsc.bitcast(dstall_v[pl.ds(off, 16)], jnp.int16)
                d0, d1 = plsc.unpack(dv, format=plsc.PackFormat.INTERLEAVED)
                dbuf[pl.ds(q * 32, 16)] = d0
                dbuf[pl.ds(q * 32 + 16, 16)] = d1

        def start_gather(j, rbuf, sbuf, dbuf, sem):
            prep_idx(j, sbuf, dbuf)
            pltpu.async_copy(y_hbm.at[sbuf], rbuf, sem)

        def drain_gather(sem, rbuf):
            pltpu.make_async_copy(y_hbm.at[pl.ds(0, _ECB)], rbuf, sem).wait()

        def process(rbuf, dvbuf):
            pltpu.sync_copy(rbuf, agg_sh.at[dvbuf], add=True)
            if with_cnt:
                pltpu.sync_copy(ones_v, cnt_sh.at[dvbuf], add=True)

        start_gather(0, rows0, src0_v, dst0_v, semg0)

        def step(k, _):
            j0 = 2 * k
            start_gather(j0 + 1, rows1, src1_v, dst1_v, semg1)
            drain_gather(semg0, rows0)
            process(rows0, dst0_v)
            start_gather(j0 + 2, rows0, src0_v, dst0_v, semg0)
            drain_gather(semg1, rows1)
            process(rows1, dst1_v)
            return 0
        lax.fori_loop(0, _EFULL // 2 - 1, step, 0)

        start_gather(_EFULL - 1, rows1, src1_v, dst1_v, semg1)
        drain_gather(semg0, rows0)
        process(rows0, dst0_v)
        drain_gather(semg1, rows1)
        process(rows1, dst1_v)

        pltpu.sync_copy(src_hbm.at[pl.ds(ebase + _ETAIL, 16)], srct_v)
        pltpu.sync_copy(dst_hbm.at[pl.ds(ebase + _ETAIL, 16)], dstt_v)
        pltpu.async_copy(
            y_hbm.at[srct_v], rows0.at[pl.ds(0, 16)], semg0).wait()
        pltpu.sync_copy(rows0.at[pl.ds(0, 16)], agg_sh.at[dstt_v], add=True)
        if with_cnt:
            pltpu.sync_copy(ones_v.at[pl.ds(0, 16)], cnt_sh.at[dstt_v], add=True)

        plsc.subcore_barrier()

        def dump(k, _):
            r0 = sid * _RPT + k * 80
            g0 = cid * _NP + r0
            pltpu.sync_copy(agg_sh.at[pl.ds(r0, 80)], rows0.at[pl.ds(0, 80)])
            pltpu.sync_copy(rows0.at[pl.ds(0, 80)], out_hbm.at[pl.ds(g0, 80)])
            if with_cnt:
                pltpu.sync_copy(cnt_sh.at[pl.ds(r0, 80)], ones_v.at[pl.ds(0, 80)])
                pltpu.sync_copy(ones_v.at[pl.ds(0, 80)], cnt_hbm.at[pl.ds(g0, 80)])
            return 0

        lax.fori_loop(0, _RPT // 80, dump, 0)

    return pl.kernel(
        body, out_type=out_type, mesh=mesh, scratch_types=scratch,
        compiler_params=pltpu.CompilerParams(needs_layout_passes=False))


_agg_l1 = _make_agg(_H, with_cnt=True)
_agg_l2 = _make_agg(_H, with_cnt=False)


def _make_decoder():
    mesh = plsc.VectorSubcoreMesh(core_axis_name="c", subcore_axis_name="s")
    out_type = jax.ShapeDtypeStruct((_LP,), jnp.float32)
    scratch = [
        pltpu.VMEM((_LPW,), jnp.int32),
        pltpu.VMEM((_LPW,), jnp.int32),
        pltpu.VMEM((_LC, _H), jnp.float32),
        pltpu.VMEM((_LC, _H), jnp.float32),
        pltpu.VMEM((_LC, _H), jnp.float32),
        pltpu.VMEM((_LC, _H), jnp.float32),
        pltpu.VMEM((_LPW,), jnp.float32),
        pltpu.VMEM((16, 16), jnp.float32),
        pltpu.VMEM_SHARED((_N, _H), jnp.float32),
        pltpu.SemaphoreType.DMA,
        pltpu.SemaphoreType.DMA,
    ]

    def body(x2_hbm, ia_hbm, ib_hbm, pred_hbm,
             iaall_v, iball_v, ra0, rb0, ra1, rb1, out_v, tp_v, tbl_sh,
             sem0, sem1):
        sid = lax.axis_index("s")
        cid = lax.axis_index("c")
        wid = cid * _NS + sid
        lbase = wid * _LPW

        cpa = pltpu.async_copy(ia_hbm.at[pl.ds(lbase, _LPW)], iaall_v, sem0)
        cpb = pltpu.async_copy(ib_hbm.at[pl.ds(lbase, _LPW)], iball_v, sem0)

        def tload(k, _):
            cidx = sid * 16 + k

            @pl.when(cidx < _N // 40)
            def _():
                r0 = cidx * 40
                pltpu.sync_copy(x2_hbm.at[pl.ds(r0, 40)], ra0.at[pl.ds(0, 40)])
                pltpu.sync_copy(ra0.at[pl.ds(0, 40)], tbl_sh.at[pl.ds(r0, 40)])
            return 0
        lax.fori_loop(0, 16, tload, 0)

        cpa.wait()
        cpb.wait()
        plsc.subcore_barrier()

        def start_pair(j, ra, rb, sem):
            pltpu.async_copy(tbl_sh.at[iaall_v.at[pl.ds(j * _LC, _LC)]], ra, sem)
            pltpu.async_copy(tbl_sh.at[iball_v.at[pl.ds(j * _LC, _LC)]], rb, sem)

        def drain_pair(sem, ra, rb):
            pltpu.make_async_copy(x2_hbm.at[pl.ds(0, _LC)], ra, sem).wait()
            pltpu.make_async_copy(x2_hbm.at[pl.ds(0, _LC)], rb, sem).wait()

        def compute(j, ra_v, rb_v):
            lane = lax.iota(jnp.int32, 16)
            for g in range(_LC // 16):
                for rr in range(16):
                    r = g * 16 + rr
                    acc = ra_v[r, pl.ds(0, 16)] * rb_v[r, pl.ds(0, 16)]
                    for q in range(1, _C // 16):
                        acc = acc + ra_v[r, pl.ds(q * 16, 16)] * rb_v[r, pl.ds(q * 16, 16)]
                    plsc.store_scatter(tp_v, [lane, jnp.full((16,), rr, jnp.int32)], acc)
                res = tp_v[0, pl.ds(0, 16)]
                for rr in range(1, 16):
                    res = res + tp_v[rr, pl.ds(0, 16)]
                out_v[pl.ds(j * _LC + g * 16, 16)] = res

        start_pair(0, ra0, rb0, sem0)

        def step(k, _):
            j0 = 2 * k
            start_pair(j0 + 1, ra1, rb1, sem1)
            drain_pair(sem0, ra0, rb0)
            compute(j0, ra0, rb0)
            start_pair(j0 + 2, ra0, rb0, sem0)
            drain_pair(sem1, ra1, rb1)
            compute(j0 + 1, ra1, rb1)
            return 0
        lax.fori_loop(0, _LNC // 2, step, 0)

        drain_pair(sem0, ra0, rb0)
        compute(_LNC - 1, ra0, rb0)

        pltpu.sync_copy(out_v, pred_hbm.at[pl.ds(lbase, _LPW)])

    return pl.kernel(
        body, out_type=out_type, mesh=mesh, scratch_types=scratch,
        compiler_params=pltpu.CompilerParams(needs_layout_passes=False))


_decoder = _make_decoder()


def kernel(node_feature, edge_index, edge_label_index, W1_l, W1_r, b1, W2_l, W2_r, b2):
    src = edge_index[0]
    dst = edge_index[1]
    src16 = lax.bitcast_convert_type(
        src.astype(jnp.int16).reshape(_E // 2, 2), jnp.int32)
    dst16 = lax.bitcast_convert_type(
        dst.astype(jnp.int16).reshape(_E // 2, 2), jnp.int32)

    y1, h1 = _tc_pre(node_feature, W1_l, W1_r, b1)
    agg1, cntp = _agg_l1(y1, src, dst, src16, dst16)
    s1 = agg1.reshape(_NC, _NP, _H)
    cnt = cntp.reshape(_NC, _NP).T

    w2lp = jnp.concatenate([W2_l, jnp.zeros((_H - _C, _H), jnp.float32)], axis=0)
    y2, h2 = _tc_mid(s1, cnt, h1, w2lp, W2_r, b2)
    agg2, = _agg_l2(y2, src, dst, src16, dst16)
    s2 = agg2.reshape(_NC, _NP, _H)

    x2 = _tc_post(s2, cnt, h2)

    pad = jnp.zeros((_LP - _L,), jnp.int32)
    ia = jnp.concatenate([edge_label_index[0], pad])
    ib = jnp.concatenate([edge_label_index[1], pad])
    pred = _decoder(x2, ia, ib)
    return pred[:_L]


# speedup vs baseline: 1.5828x; 1.5828x over previous
import functools

import jax
import jax.numpy as jnp
from jax import lax
from jax.experimental import pallas as pl
from jax.experimental.pallas import tpu as pltpu
from jax.experimental.pallas import tpu_sc as plsc

_N = 10000
_E = 320000
_L = 100000
_D = 128
_H = 128
_C = 64

_NC = 2
_NS = 16
_NW = _NC * _NS

_NP = 10240
_RPT = _NP // _NS

_EPW = _E // _NW
_ECB = 64
_EFULL = 156
_ETAIL = _EFULL * _ECB

_CNTW = 16

_LP = 100352
_LPW = _LP // _NW
_LC = 64
_LNC = _LPW // _LC

_ROWB = 25
_RB = _N // _ROWB


def _dotT(x, w):
    return lax.dot_general(x, w, (((1,), (1,)), ((), ())),
                           preferred_element_type=jnp.float32)


def _tc_pre(x, w1l, w1r, b1):
    def body(x_ref, wl_ref, wr_ref, b_ref, y_ref, h_ref):
        xb = x_ref[...]
        y_ref[...] = _dotT(xb, wl_ref[...])
        h_ref[...] = _dotT(xb, wr_ref[...]) + b_ref[...]
    return pl.pallas_call(
        body,
        grid=(_ROWB,),
        in_specs=[
            pl.BlockSpec((_RB, _D), lambda i: (i, 0)),
            pl.BlockSpec((_H, _D), lambda i: (0, 0)),
            pl.BlockSpec((_H, _D), lambda i: (0, 0)),
            pl.BlockSpec((1, _H), lambda i: (0, 0)),
        ],
        out_specs=[pl.BlockSpec((_RB, _H), lambda i: (i, 0))] * 2,
        out_shape=[jax.ShapeDtypeStruct((_N, _H), jnp.float32)] * 2,
    )(x, w1l, w1r, b1.reshape(1, _H))


def _tc_mid(s1, cnt, h1, w2lp, w2r, b2):
    def body(s_ref, c_ref, h_ref, wl_ref, wr_ref, b_ref, y_ref, h2_ref):
        s = s_ref[0] + s_ref[1]
        c = c_ref[:, 0:1] + c_ref[:, 1:2]
        x1 = jnp.maximum(s / jnp.maximum(c, 1.0) + h_ref[...], 0.0)
        y_ref[...] = _dotT(x1, wl_ref[...])
        h2_ref[...] = _dotT(x1, wr_ref[...]) + b_ref[...]
    return pl.pallas_call(
        body,
        grid=(_ROWB,),
        in_specs=[
            pl.BlockSpec((_NC, _RB, _H), lambda i: (0, i, 0)),
            pl.BlockSpec((_RB, _NC), lambda i: (i, 0)),
            pl.BlockSpec((_RB, _H), lambda i: (i, 0)),
            pl.BlockSpec((_H, _H), lambda i: (0, 0)),
            pl.BlockSpec((_C, _H), lambda i: (0, 0)),
            pl.BlockSpec((1, _C), lambda i: (0, 0)),
        ],
        out_specs=[pl.BlockSpec((_RB, _H), lambda i: (i, 0)),
                   pl.BlockSpec((_RB, _C), lambda i: (i, 0))],
        out_shape=[jax.ShapeDtypeStruct((_N, _H), jnp.float32),
                   jax.ShapeDtypeStruct((_N, _C), jnp.float32)],
    )(s1, cnt, h1, w2lp, w2r, b2.reshape(1, _C))


def _tc_post(s2, cnt, h2):
    def body(s_ref, c_ref, h_ref, x_ref):
        s = s_ref[0][:, 0:_C] + s_ref[1][:, 0:_C]
        c = c_ref[:, 0:1] + c_ref[:, 1:2]
        x2 = s / jnp.maximum(c, 1.0) + h_ref[...]
        x_ref[...] = jnp.concatenate(
            [x2, jnp.zeros((_RB, _H - _C), jnp.float32)], axis=1)
    return pl.pallas_call(
        body,
        grid=(_ROWB,),
        in_specs=[
            pl.BlockSpec((_NC, _RB, _H), lambda i: (0, i, 0)),
            pl.BlockSpec((_RB, _NC), lambda i: (i, 0)),
            pl.BlockSpec((_RB, _C), lambda i: (i, 0)),
        ],
        out_specs=pl.BlockSpec((_RB, _H), lambda i: (i, 0)),
        out_shape=jax.ShapeDtypeStruct((_N, _H), jnp.float32),
    )(s2, cnt, h2)


def _zero_vmem(ref, nrow, width):
    def zrow(i, _):
        for q in range(width // 16):
            ref[i, pl.ds(q * 16, 16)] = jnp.zeros((16,), jnp.float32)
        return 0
    lax.fori_loop(0, nrow, zrow, 0)


def _make_agg(F, with_cnt):
    mesh = plsc.VectorSubcoreMesh(core_axis_name="c", subcore_axis_name="s")
    out_type = [jax.ShapeDtypeStruct((_NC * _NP, F), jnp.float32)]
    scratch = [
        pltpu.VMEM((_EPW,), jnp.int32),
        pltpu.VMEM((_EPW,), jnp.int32),
        pltpu.VMEM((_ECB,), jnp.int32),
        pltpu.VMEM((_ECB,), jnp.int32),
        pltpu.VMEM((16,), jnp.int32),
        pltpu.VMEM((_ECB, F), jnp.float32),
        pltpu.VMEM((_ECB, F), jnp.float32),
        pltpu.VMEM_SHARED((_NP, F), jnp.float32),
        pltpu.SemaphoreType.DMA,
        pltpu.SemaphoreType.DMA,
    ]
    if with_cnt:
        out_type.append(jax.ShapeDtypeStruct((_NC * _NP,), jnp.float32))
        scratch += [
            pltpu.VMEM((_ECB,), jnp.float32),
            pltpu.VMEM_SHARED((_NP,), jnp.float32),
        ]

    def body(y_hbm, src_hbm, dst_hbm, *rest):
        if with_cnt:
            (out_hbm, cnt_hbm, srcall_v, dstall_v,
             dst0_v, dst1_v, dstt_v,
             rows0, rows1, agg_sh, semg0, semg1, ones_v, cnt_sh) = rest
        else:
            (out_hbm, srcall_v, dstall_v,
             dst0_v, dst1_v, dstt_v,
             rows0, rows1, agg_sh, semg0, semg1) = rest
        cid = lax.axis_index("c")
        sid = lax.axis_index("s")
        wid = cid * _NS + sid
        ebase = wid * _EPW

        cpa = pltpu.async_copy(src_hbm.at[pl.ds(ebase, _EPW)], srcall_v, semg0)
        cpb = pltpu.async_copy(dst_hbm.at[pl.ds(ebase, _EPW)], dstall_v, semg0)

        _zero_vmem(rows0, _ECB, F)
        if with_cnt:
            for q in range(_ECB // 16):
                ones_v[pl.ds(q * 16, 16)] = jnp.zeros((16,), jnp.float32)

        def zslab(k, _):
            r0 = sid * _RPT + k * _ECB
            pltpu.sync_copy(rows0, agg_sh.at[pl.ds(r0, _ECB)])
            if with_cnt:
                pltpu.sync_copy(ones_v, cnt_sh.at[pl.ds(r0, _ECB)])
            return 0
        lax.fori_loop(0, _RPT // _ECB, zslab, 0)

        if with_cnt:
            for q in range(_ECB // 16):
                ones_v[pl.ds(q * 16, 16)] = jnp.ones((16,), jnp.float32)

        cpa.wait()
        cpb.wait()
        plsc.subcore_barrier()

        def start_gather(j, rbuf, sem):
            pltpu.async_copy(
                y_hbm.at[srcall_v.at[pl.ds(j * _ECB, _ECB)]], rbuf, sem)

        def drain_gather(sem, rbuf):
            pltpu.make_async_copy(y_hbm.at[pl.ds(0, _ECB)], rbuf, sem).wait()

        def process(j, rbuf, dvbuf):
            for q in range(_ECB // 16):
                dvbuf[pl.ds(q * 16, 16)] = dstall_v[pl.ds(j * _ECB + q * 16, 16)]
            pltpu.sync_copy(rbuf, agg_sh.at[dvbuf], add=True)
            if with_cnt:
                pltpu.sync_copy(ones_v, cnt_sh.at[dvbuf], add=True)

        start_gather(0, rows0, semg0)

        def step(k, _):
            j0 = 2 * k
            start_gather(j0 + 1, rows1, semg1)
            drain_gather(semg0, rows0)
            process(j0, rows0, dst0_v)
            start_gather(j0 + 2, rows0, semg0)
            drain_gather(semg1, rows1)
            process(j0 + 1, rows1, dst1_v)
            return 0
        lax.fori_loop(0, _EFULL // 2 - 1, step, 0)

        start_gather(_EFULL - 1, rows1, semg1)
        drain_gather(semg0, rows0)
        process(_EFULL - 2, rows0, dst0_v)
        drain_gather(semg1, rows1)
        process(_EFULL - 1, rows1, dst1_v)

        pltpu.async_copy(
            y_hbm.at[srcall_v.at[pl.ds(_ETAIL, 16)]],
            rows0.at[pl.ds(0, 16)], semg0).wait()
        dstt_v[pl.ds(0, 16)] = dstall_v[pl.ds(_ETAIL, 16)]
        pltpu.sync_copy(rows0.at[pl.ds(0, 16)], agg_sh.at[dstt_v], add=True)
        if with_cnt:
            pltpu.sync_copy(ones_v.at[pl.ds(0, 16)], cnt_sh.at[dstt_v], add=True)

        plsc.subcore_barrier()

        def dump(k, _):
            r0 = sid * _RPT + k * _ECB
            g0 = cid * _NP + r0
            pltpu.sync_copy(agg_sh.at[pl.ds(r0, _ECB)], rows0)
            pltpu.sync_copy(rows0, out_hbm.at[pl.ds(g0, _ECB)])
            if with_cnt:
                pltpu.sync_copy(cnt_sh.at[pl.ds(r0, _ECB)], ones_v)
                pltpu.sync_copy(ones_v, cnt_hbm.at[pl.ds(g0, _ECB)])
            return 0

        lax.fori_loop(0, _RPT // _ECB, dump, 0)

    return pl.kernel(
        body, out_type=out_type, mesh=mesh, scratch_types=scratch,
        compiler_params=pltpu.CompilerParams(needs_layout_passes=False))


_agg_l1 = _make_agg(_H, with_cnt=True)
_agg_l2 = _make_agg(_H, with_cnt=False)


def _make_decoder():
    mesh = plsc.VectorSubcoreMesh(core_axis_name="c", subcore_axis_name="s")
    out_type = jax.ShapeDtypeStruct((_LP,), jnp.float32)
    scratch = [
        pltpu.VMEM((_LPW,), jnp.int32),
        pltpu.VMEM((_LPW,), jnp.int32),
        pltpu.VMEM((_LC, _H), jnp.float32),
        pltpu.VMEM((_LC, _H), jnp.float32),
        pltpu.VMEM((_LC, _H), jnp.float32),
        pltpu.VMEM((_LC, _H), jnp.float32),
        pltpu.VMEM((_LPW,), jnp.float32),
        pltpu.VMEM((16, 16), jnp.float32),
        pltpu.VMEM_SHARED((_N, _H), jnp.float32),
        pltpu.SemaphoreType.DMA,
        pltpu.SemaphoreType.DMA,
    ]

    def body(x2_hbm, ia_hbm, ib_hbm, pred_hbm,
             iaall_v, iball_v, ra0, rb0, ra1, rb1, out_v, tp_v, tbl_sh,
             sem0, sem1):
        sid = lax.axis_index("s")
        cid = lax.axis_index("c")
        wid = cid * _NS + sid
        lbase = wid * _LPW

        cpa = pltpu.async_copy(ia_hbm.at[pl.ds(lbase, _LPW)], iaall_v, sem0)
        cpb = pltpu.async_copy(ib_hbm.at[pl.ds(lbase, _LPW)], iball_v, sem0)

        def tload(k, _):
            cidx = sid * 16 + k

            @pl.when(cidx < _N // 40)
            def _():
                r0 = cidx * 40
                pltpu.sync_copy(x2_hbm.at[pl.ds(r0, 40)], ra0.at[pl.ds(0, 40)])
                pltpu.sync_copy(ra0.at[pl.ds(0, 40)], tbl_sh.at[pl.ds(r0, 40)])
            return 0
        lax.fori_loop(0, 16, tload, 0)

        cpa.wait()
        cpb.wait()
        plsc.subcore_barrier()

        def start_pair(j, ra, rb, sem):
            pltpu.async_copy(tbl_sh.at[iaall_v.at[pl.ds(j * _LC, _LC)]], ra, sem)
            pltpu.async_copy(tbl_sh.at[iball_v.at[pl.ds(j * _LC, _LC)]], rb, sem)

        def drain_pair(sem, ra, rb):
            pltpu.make_async_copy(x2_hbm.at[pl.ds(0, _LC)], ra, sem).wait()
            pltpu.make_async_copy(x2_hbm.at[pl.ds(0, _LC)], rb, sem).wait()

        def compute(j, ra_v, rb_v):
            lane = lax.iota(jnp.int32, 16)
            for g in range(_LC // 16):
                for rr in range(16):
                    r = g * 16 + rr
                    acc = ra_v[r, pl.ds(0, 16)] * rb_v[r, pl.ds(0, 16)]
                    for q in range(1, _C // 16):
                        acc = acc + ra_v[r, pl.ds(q * 16, 16)] * rb_v[r, pl.ds(q * 16, 16)]
                    plsc.store_scatter(tp_v, [lane, jnp.full((16,), rr, jnp.int32)], acc)
                res = tp_v[0, pl.ds(0, 16)]
                for rr in range(1, 16):
                    res = res + tp_v[rr, pl.ds(0, 16)]
                out_v[pl.ds(j * _LC + g * 16, 16)] = res

        start_pair(0, ra0, rb0, sem0)

        def step(k, _):
            j0 = 2 * k
            start_pair(j0 + 1, ra1, rb1, sem1)
            drain_pair(sem0, ra0, rb0)
            compute(j0, ra0, rb0)
            start_pair(j0 + 2, ra0, rb0, sem0)
            drain_pair(sem1, ra1, rb1)
            compute(j0 + 1, ra1, rb1)
            return 0
        lax.fori_loop(0, _LNC // 2, step, 0)

        drain_pair(sem0, ra0, rb0)
        compute(_LNC - 1, ra0, rb0)

        pltpu.sync_copy(out_v, pred_hbm.at[pl.ds(lbase, _LPW)])

    return pl.kernel(
        body, out_type=out_type, mesh=mesh, scratch_types=scratch,
        compiler_params=pltpu.CompilerParams(needs_layout_passes=False))


_decoder = _make_decoder()


def kernel(node_feature, edge_index, edge_label_index, W1_l, W1_r, b1, W2_l, W2_r, b2):
    src = edge_index[0]
    dst = edge_index[1]
    y1, h1 = _tc_pre(node_feature, W1_l, W1_r, b1)
    agg1, cntp = _agg_l1(y1, src, dst)
    s1 = agg1.reshape(_NC, _NP, _H)
    cnt = cntp.reshape(_NC, _NP).T

    w2lp = jnp.concatenate([W2_l, jnp.zeros((_H - _C, _H), jnp.float32)], axis=0)
    y2, h2 = _tc_mid(s1, cnt, h1, w2lp, W2_r, b2)
    agg2, = _agg_l2(y2, src, dst)
    s2 = agg2.reshape(_NC, _NP, _H)

    x2 = _tc_post(s2, cnt, h2)

    pad = jnp.zeros((_LP - _L,), jnp.int32)
    ia = jnp.concatenate([edge_label_index[0], pad])
    ib = jnp.concatenate([edge_label_index[1], pad])
    pred = _decoder(x2, ia, ib)
    return pred[:_L]


# speedup vs baseline: 1.8424x; 1.1640x over previous
import functools

import jax
import jax.numpy as jnp
from jax import lax
from jax.experimental import pallas as pl
from jax.experimental.pallas import tpu as pltpu
from jax.experimental.pallas import tpu_sc as plsc

_N = 10000
_E = 320000
_L = 100000
_D = 128
_H = 128
_C = 64

_NC = 2
_NS = 16
_NW = _NC * _NS

_NP = 10240
_RPT = _NP // _NS

_EPW = _E // _NW
_ECB = 64
_EFULL = 156
_ETAIL = _EFULL * _ECB

_CNTW = 16

_LP = 100352
_LPW = _LP // _NW
_LC = 64
_LNC = _LPW // _LC

_ROWB = 25
_RB = _N // _ROWB


def _dotT(x, w):
    return lax.dot_general(x, w, (((1,), (1,)), ((), ())),
                           preferred_element_type=jnp.float32)


def _tc_pre(x, w1l, w1r, b1):
    def body(x_ref, wl_ref, wr_ref, b_ref, y_ref, h_ref):
        xb = x_ref[...]
        y_ref[...] = _dotT(xb, wl_ref[...])
        h_ref[...] = _dotT(xb, wr_ref[...]) + b_ref[...]
    return pl.pallas_call(
        body,
        grid=(_ROWB,),
        in_specs=[
            pl.BlockSpec((_RB, _D), lambda i: (i, 0)),
            pl.BlockSpec((_H, _D), lambda i: (0, 0)),
            pl.BlockSpec((_H, _D), lambda i: (0, 0)),
            pl.BlockSpec((1, _H), lambda i: (0, 0)),
        ],
        out_specs=[pl.BlockSpec((_RB, _H), lambda i: (i, 0))] * 2,
        out_shape=[jax.ShapeDtypeStruct((_N, _H), jnp.float32)] * 2,
    )(x, w1l, w1r, b1.reshape(1, _H))


def _tc_mid(s1, cnt, h1, w2lp, w2r, b2):
    def body(s_ref, c_ref, h_ref, wl_ref, wr_ref, b_ref, y_ref, h2_ref):
        s = s_ref[0] + s_ref[1]
        c = c_ref[:, 0:1] + c_ref[:, 1:2]
        x1 = jnp.maximum(s / jnp.maximum(c, 1.0) + h_ref[...], 0.0)
        y_ref[...] = _dotT(x1, wl_ref[...])
        h2_ref[...] = _dotT(x1, wr_ref[...]) + b_ref[...]
    return pl.pallas_call(
        body,
        grid=(_ROWB,),
        in_specs=[
            pl.BlockSpec((_NC, _RB, _H), lambda i: (0, i, 0)),
            pl.BlockSpec((_RB, _NC), lambda i: (i, 0)),
            pl.BlockSpec((_RB, _H), lambda i: (i, 0)),
            pl.BlockSpec((_H, _H), lambda i: (0, 0)),
            pl.BlockSpec((_C, _H), lambda i: (0, 0)),
            pl.BlockSpec((1, _C), lambda i: (0, 0)),
        ],
        out_specs=[pl.BlockSpec((_RB, _H), lambda i: (i, 0)),
                   pl.BlockSpec((_RB, _C), lambda i: (i, 0))],
        out_shape=[jax.ShapeDtypeStruct((_N, _H), jnp.float32),
                   jax.ShapeDtypeStruct((_N, _C), jnp.float32)],
    )(s1, cnt, h1, w2lp, w2r, b2.reshape(1, _C))


def _tc_post(s2, cnt, h2):
    def body(s_ref, c_ref, h_ref, x_ref):
        s = s_ref[0][:, 0:_C] + s_ref[1][:, 0:_C]
        c = c_ref[:, 0:1] + c_ref[:, 1:2]
        x2 = s / jnp.maximum(c, 1.0) + h_ref[...]
        x_ref[...] = jnp.concatenate(
            [x2, jnp.zeros((_RB, _H - _C), jnp.float32)], axis=1)
    return pl.pallas_call(
        body,
        grid=(_ROWB,),
        in_specs=[
            pl.BlockSpec((_NC, _RB, _H), lambda i: (0, i, 0)),
            pl.BlockSpec((_RB, _NC), lambda i: (i, 0)),
            pl.BlockSpec((_RB, _C), lambda i: (i, 0)),
        ],
        out_specs=pl.BlockSpec((_RB, _H), lambda i: (i, 0)),
        out_shape=jax.ShapeDtypeStruct((_N, _H), jnp.float32),
    )(s2, cnt, h2)


def _zero_vmem(ref, nrow, width):
    def zrow(i, _):
        for q in range(width // 16):
            ref[i, pl.ds(q * 16, 16)] = jnp.zeros((16,), jnp.float32)
        return 0
    lax.fori_loop(0, nrow, zrow, 0)


def _make_agg(F, with_cnt):
    mesh = plsc.VectorSubcoreMesh(core_axis_name="c", subcore_axis_name="s")
    out_type = [jax.ShapeDtypeStruct((_NC * _NP, F), jnp.float32)]
    scratch = [
        pltpu.VMEM((_EPW,), jnp.int32),
        pltpu.VMEM((_ECB,), jnp.int32),
        pltpu.VMEM((_ECB,), jnp.int32),
        pltpu.VMEM((_ECB,), jnp.int32),
        pltpu.VMEM((16,), jnp.int32),
        pltpu.VMEM((_ECB, F), jnp.float32),
        pltpu.VMEM((_ECB, F), jnp.float32),
        pltpu.VMEM((_ECB, F), jnp.float32),
        pltpu.VMEM_SHARED((_NP, F), jnp.float32),
        pltpu.SemaphoreType.DMA,
        pltpu.SemaphoreType.DMA,
        pltpu.SemaphoreType.DMA,
    ]
    if with_cnt:
        out_type.append(jax.ShapeDtypeStruct((_NC * _NP,), jnp.float32))
        scratch += [
            pltpu.VMEM((_ECB,), jnp.float32),
            pltpu.VMEM_SHARED((_NP,), jnp.float32),
        ]

    def body(y_hbm, src_hbm, dst_hbm, *rest):
        if with_cnt:
            (out_hbm, cnt_hbm, srcall_v,
             dst0_v, dst1_v, dst2_v, dstt_v,
             rows0, rows1, rows2, agg_sh,
             semg0, semg1, semg2, ones_v, cnt_sh) = rest
        else:
            (out_hbm, srcall_v,
             dst0_v, dst1_v, dst2_v, dstt_v,
             rows0, rows1, rows2, agg_sh,
             semg0, semg1, semg2) = rest
        cid = lax.axis_index("c")
        sid = lax.axis_index("s")
        wid = cid * _NS + sid
        ebase = wid * _EPW

        cpa = pltpu.async_copy(src_hbm.at[pl.ds(ebase, _EPW)], srcall_v, semg0)

        _zero_vmem(rows0, _ECB, F)
        if with_cnt:
            for q in range(_ECB // 16):
                ones_v[pl.ds(q * 16, 16)] = jnp.zeros((16,), jnp.float32)

        def zslab(k, _):
            r0 = sid * _RPT + k * _ECB
            pltpu.sync_copy(rows0, agg_sh.at[pl.ds(r0, _ECB)])
            if with_cnt:
                pltpu.sync_copy(ones_v, cnt_sh.at[pl.ds(r0, _ECB)])
            return 0
        lax.fori_loop(0, _RPT // _ECB, zslab, 0)

        if with_cnt:
            for q in range(_ECB // 16):
                ones_v[pl.ds(q * 16, 16)] = jnp.ones((16,), jnp.float32)

        cpa.wait()
        plsc.subcore_barrier()

        def start(j, rbuf, dvbuf, sem):
            pltpu.async_copy(
                dst_hbm.at[pl.ds(ebase + j * _ECB, _ECB)], dvbuf, sem)
            pltpu.async_copy(
                y_hbm.at[srcall_v.at[pl.ds(j * _ECB, _ECB)]], rbuf, sem)

        def drain(sem, rbuf, dvbuf):
            pltpu.make_async_copy(y_hbm.at[pl.ds(0, _ECB)], rbuf, sem).wait()
            pltpu.make_async_copy(dst_hbm.at[pl.ds(0, _ECB)], dvbuf, sem).wait()

        def process(rbuf, dvbuf):
            pltpu.sync_copy(rbuf, agg_sh.at[dvbuf], add=True)
            if with_cnt:
                pltpu.sync_copy(ones_v, cnt_sh.at[dvbuf], add=True)

        start(0, rows0, dst0_v, semg0)
        start(1, rows1, dst1_v, semg1)

        def step(k, _):
            j0 = 3 * k
            start(j0 + 2, rows2, dst2_v, semg2)
            drain(semg0, rows0, dst0_v)
            process(rows0, dst0_v)
            start(j0 + 3, rows0, dst0_v, semg0)
            drain(semg1, rows1, dst1_v)
            process(rows1, dst1_v)
            start(j0 + 4, rows1, dst1_v, semg1)
            drain(semg2, rows2, dst2_v)
            process(rows2, dst2_v)
            return 0
        lax.fori_loop(0, _EFULL // 3 - 2, step, 0)

        start(_EFULL - 4, rows2, dst2_v, semg2)
        drain(semg0, rows0, dst0_v)
        process(rows0, dst0_v)
        start(_EFULL - 3, rows0, dst0_v, semg0)
        drain(semg1, rows1, dst1_v)
        process(rows1, dst1_v)
        start(_EFULL - 2, rows1, dst1_v, semg1)
        drain(semg2, rows2, dst2_v)
        process(rows2, dst2_v)
        start(_EFULL - 1, rows2, dst2_v, semg2)
        drain(semg0, rows0, dst0_v)
        process(rows0, dst0_v)
        drain(semg1, rows1, dst1_v)
        process(rows1, dst1_v)
        drain(semg2, rows2, dst2_v)
        process(rows2, dst2_v)

        pltpu.sync_copy(dst_hbm.at[pl.ds(ebase + _ETAIL, 16)], dstt_v)
        pltpu.async_copy(
            y_hbm.at[srcall_v.at[pl.ds(_ETAIL, 16)]],
            rows0.at[pl.ds(0, 16)], semg0).wait()
        pltpu.sync_copy(rows0.at[pl.ds(0, 16)], agg_sh.at[dstt_v], add=True)
        if with_cnt:
            pltpu.sync_copy(ones_v.at[pl.ds(0, 16)], cnt_sh.at[dstt_v], add=True)

        plsc.subcore_barrier()

        def dump(k, _):
            r0 = sid * _RPT + k * _ECB
            g0 = cid * _NP + r0
            pltpu.sync_copy(agg_sh.at[pl.ds(r0, _ECB)], rows0)
            pltpu.sync_copy(rows0, out_hbm.at[pl.ds(g0, _ECB)])
            if with_cnt:
                pltpu.sync_copy(cnt_sh.at[pl.ds(r0, _ECB)], ones_v)
                pltpu.sync_copy(ones_v, cnt_hbm.at[pl.ds(g0, _ECB)])
            return 0

        lax.fori_loop(0, _RPT // _ECB, dump, 0)

    return pl.kernel(
        body, out_type=out_type, mesh=mesh, scratch_types=scratch,
        compiler_params=pltpu.CompilerParams(needs_layout_passes=False))


_agg_l1 = _make_agg(_H, with_cnt=True)
_agg_l2 = _make_agg(_H, with_cnt=False)


def _make_decoder():
    mesh = plsc.VectorSubcoreMesh(core_axis_name="c", subcore_axis_name="s")
    out_type = jax.ShapeDtypeStruct((_LP,), jnp.float32)
    scratch = [
        pltpu.VMEM((_LPW,), jnp.int32),
        pltpu.VMEM((_LPW,), jnp.int32),
        pltpu.VMEM((_LC, _H), jnp.float32),
        pltpu.VMEM((_LC, _H), jnp.float32),
        pltpu.VMEM((_LC, _H), jnp.float32),
        pltpu.VMEM((_LC, _H), jnp.float32),
        pltpu.VMEM((_LPW,), jnp.float32),
        pltpu.VMEM((16, 16), jnp.float32),
        pltpu.VMEM_SHARED((_N, _H), jnp.float32),
        pltpu.SemaphoreType.DMA,
        pltpu.SemaphoreType.DMA,
    ]

    def body(x2_hbm, ia_hbm, ib_hbm, pred_hbm,
             iaall_v, iball_v, ra0, rb0, ra1, rb1, out_v, tp_v, tbl_sh,
             sem0, sem1):
        sid = lax.axis_index("s")
        cid = lax.axis_index("c")
        wid = cid * _NS + sid
        lbase = wid * _LPW

        cpa = pltpu.async_copy(ia_hbm.at[pl.ds(lbase, _LPW)], iaall_v, sem0)
        cpb = pltpu.async_copy(ib_hbm.at[pl.ds(lbase, _LPW)], iball_v, sem0)

        def tload(k, _):
            cidx = sid * 16 + k

            @pl.when(cidx < _N // 40)
            def _():
                r0 = cidx * 40
                pltpu.sync_copy(x2_hbm.at[pl.ds(r0, 40)], ra0.at[pl.ds(0, 40)])
                pltpu.sync_copy(ra0.at[pl.ds(0, 40)], tbl_sh.at[pl.ds(r0, 40)])
            return 0
        lax.fori_loop(0, 16, tload, 0)

        cpa.wait()
        cpb.wait()
        plsc.subcore_barrier()

        def start_pair(j, ra, rb, sem):
            pltpu.async_copy(tbl_sh.at[iaall_v.at[pl.ds(j * _LC, _LC)]], ra, sem)
            pltpu.async_copy(tbl_sh.at[iball_v.at[pl.ds(j * _LC, _LC)]], rb, sem)

        def drain_pair(sem, ra, rb):
            pltpu.make_async_copy(x2_hbm.at[pl.ds(0, _LC)], ra, sem).wait()
            pltpu.make_async_copy(x2_hbm.at[pl.ds(0, _LC)], rb, sem).wait()

        def compute(j, ra_v, rb_v):
            lane = lax.iota(jnp.int32, 16)
            for g in range(_LC // 16):
                for rr in range(16):
                    r = g * 16 + rr
                    acc = ra_v[r, pl.ds(0, 16)] * rb_v[r, pl.ds(0, 16)]
                    for q in range(1, _C // 16):
                        acc = acc + ra_v[r, pl.ds(q * 16, 16)] * rb_v[r, pl.ds(q * 16, 16)]
                    plsc.store_scatter(tp_v, [lane, jnp.full((16,), rr, jnp.int32)], acc)
                res = tp_v[0, pl.ds(0, 16)]
                for rr in range(1, 16):
                    res = res + tp_v[rr, pl.ds(0, 16)]
                out_v[pl.ds(j * _LC + g * 16, 16)] = res

        start_pair(0, ra0, rb0, sem0)

        def step(k, _):
            j0 = 2 * k
            start_pair(j0 + 1, ra1, rb1, sem1)
            drain_pair(sem0, ra0, rb0)
            compute(j0, ra0, rb0)
            start_pair(j0 + 2, ra0, rb0, sem0)
            drain_pair(sem1, ra1, rb1)
            compute(j0 + 1, ra1, rb1)
            return 0
        lax.fori_loop(0, _LNC // 2, step, 0)

        drain_pair(sem0, ra0, rb0)
        compute(_LNC - 1, ra0, rb0)

        pltpu.sync_copy(out_v, pred_hbm.at[pl.ds(lbase, _LPW)])

    return pl.kernel(
        body, out_type=out_type, mesh=mesh, scratch_types=scratch,
        compiler_params=pltpu.CompilerParams(needs_layout_passes=False))


_decoder = _make_decoder()


def kernel(node_feature, edge_index, edge_label_index, W1_l, W1_r, b1, W2_l, W2_r, b2):
    src = edge_index[0]
    dst = edge_index[1]
    y1, h1 = _tc_pre(node_feature, W1_l, W1_r, b1)
    agg1, cntp = _agg_l1(y1, src, dst)
    s1 = agg1.reshape(_NC, _NP, _H)
    cnt = cntp.reshape(_NC, _NP).T

    w2lp = jnp.concatenate([W2_l, jnp.zeros((_H - _C, _H), jnp.float32)], axis=0)
    y2, h2 = _tc_mid(s1, cnt, h1, w2lp, W2_r, b2)
    agg2, = _agg_l2(y2, src, dst)
    s2 = agg2.reshape(_NC, _NP, _H)

    x2 = _tc_post(s2, cnt, h2)

    pad = jnp.zeros((_LP - _L,), jnp.int32)
    ia = jnp.concatenate([edge_label_index[0], pad])
    ib = jnp.concatenate([edge_label_index[1], pad])
    pred = _decoder(x2, ia, ib)
    return pred[:_L]
